# baseline (speedup 1.0000x reference)
import functools
import math

import jax
import jax.numpy as jnp
from jax import lax
from jax.experimental import pallas as pl
from jax.experimental.pallas import tpu as pltpu
from jax.experimental.pallas import tpu_sc as plsc

N = 10000
E = 320000
D = 128
NC = 2
NS = 16
L = 16
NW = NC * NS
NPAD = 10240
EPAD = 327680
EPW = EPAD // NW
C = 128
NCH = EPW // C
SLICE = NPAD // NS
NEG = -3.0e38
SCALE = 1.0 / math.sqrt(float(D))

_mesh = functools.partial(
    plsc.VectorSubcoreMesh,
    core_axis_name="c", subcore_axis_name="s", num_cores=NC, num_subcores=NS)


def _lane_sum(a, lanes):
    for k in (8, 4, 2, 1):
        perm = (lanes + k) & (L - 1)
        a = a + jnp.take_along_axis(a, perm, axis=0)
    return a


def _proj4_call(h, Wq, bq, Wk, bk, Wv, bv, Ws, bs, relu_in=False,
                a0=None, a1=None, den=None):
    BP = 1024
    grid = (NPAD // BP,)
    row_spec = pl.BlockSpec((BP, D), lambda i: (i, 0))
    w_spec = pl.BlockSpec((D, D), lambda i: (0, 0))
    b_spec = pl.BlockSpec((1, D), lambda i: (0, 0))
    den_spec = pl.BlockSpec((BP, NW), lambda i: (i, 0))

    combine = a0 is not None

    def body(*refs):
        if combine:
            a0r, a1r, dr, skr, wqr, bqr, wkr, bkr, wvr, bvr, wsr, bsr, \
                qo, ko, vo, so = refs
            dd = jnp.sum(dr[...], axis=1, keepdims=True) + 1e-16
            hb = (a0r[...] + a1r[...]) / dd + skr[...]
            if relu_in:
                hb = jnp.maximum(hb, 0.0)
        else:
            hr, wqr, bqr, wkr, bkr, wvr, bvr, wsr, bsr, qo, ko, vo, so = refs
            hb = hr[...]
        qo[...] = jnp.dot(hb, wqr[...], preferred_element_type=jnp.float32) + bqr[...]
        ko[...] = jnp.dot(hb, wkr[...], preferred_element_type=jnp.float32) + bkr[...]
        vo[...] = jnp.dot(hb, wvr[...], preferred_element_type=jnp.float32) + bvr[...]
        so[...] = jnp.dot(hb, wsr[...], preferred_element_type=jnp.float32) + bsr[...]

    if combine:
        in_specs = [row_spec, row_spec, den_spec, row_spec] + \
                   [w_spec, b_spec] * 4
        args = (a0, a1, den, h,
                Wq, bq.reshape(1, D), Wk, bk.reshape(1, D),
                Wv, bv.reshape(1, D), Ws, bs.reshape(1, D))
    else:
        in_specs = [row_spec] + [w_spec, b_spec] * 4
        args = (h, Wq, bq.reshape(1, D), Wk, bk.reshape(1, D),
                Wv, bv.reshape(1, D), Ws, bs.reshape(1, D))

    out = jax.ShapeDtypeStruct((NPAD, D), jnp.float32)
    return pl.pallas_call(
        body, grid=grid, in_specs=in_specs,
        out_specs=[row_spec] * 4, out_shape=[out] * 4,
    )(*args)


def _finalize_call(a0, a1, den, sk, Wl1, bl1, Wl2, bl2, Wr1, br1, Wr2, br2):
    BP = 1024
    grid = (NPAD // BP,)
    row_spec = pl.BlockSpec((BP, D), lambda i: (i, 0))
    w_spec = pl.BlockSpec((D, D), lambda i: (0, 0))
    b_spec = pl.BlockSpec((1, D), lambda i: (0, 0))
    w2_spec = pl.BlockSpec((D, 1), lambda i: (0, 0))
    b2_spec = pl.BlockSpec((1, 1), lambda i: (0, 0))
    col_spec = pl.BlockSpec((BP, 1), lambda i: (i, 0))

    den_spec = pl.BlockSpec((BP, NW), lambda i: (i, 0))

    def body(a0r, a1r, dr, skr, wl1, bl1r, wl2, bl2r,
             wr1, br1r, wr2, br2r, so):
        dd = jnp.sum(dr[...], axis=1, keepdims=True) + 1e-16
        f = (a0r[...] + a1r[...]) / dd + skr[...]
        tl = jnp.dot(f, wl1[...], preferred_element_type=jnp.float32) + bl1r[...]
        al = jnp.dot(tl, wl2[...], preferred_element_type=jnp.float32) + bl2r[...]
        tr = jnp.dot(f, wr1[...], preferred_element_type=jnp.float32) + br1r[...]
        ar = jnp.dot(tr, wr2[...], preferred_element_type=jnp.float32) + br2r[...]
        so[...] = (al + ar) * 0.5

    return pl.pallas_call(
        body, grid=grid,
        in_specs=[row_spec, row_spec, den_spec, row_spec,
                  w_spec, b_spec, w2_spec, b2_spec,
                  w_spec, b_spec, w2_spec, b2_spec],
        out_specs=col_spec,
        out_shape=jax.ShapeDtypeStruct((NPAD, 1), jnp.float32),
    )(a0, a1, den, sk,
      Wl1, bl1.reshape(1, D), Wl2, bl2.reshape(1, 1),
      Wr1, br1.reshape(1, D), Wr2, br2.reshape(1, 1))


def _outer_call(s_row, s_col):
    BR = 256
    grid = (pl.cdiv(N, BR),)

    def body(sr, sc, o):
        z = sr[...] + sc[...]
        o[...] = 1.0 / (1.0 + jnp.exp(-z))

    return pl.pallas_call(
        body, grid=grid,
        in_specs=[pl.BlockSpec((BR, 1), lambda i: (i, 0)),
                  pl.BlockSpec((1, N), lambda i: (0, 0))],
        out_specs=pl.BlockSpec((BR, N), lambda i: (i, 0)),
        out_shape=jax.ShapeDtypeStruct((N, N), jnp.float32),
    )(s_row, s_col)


def _sc_logits_max(q, k, srcp, dstp, lane_iota):
    @functools.partial(
        pl.kernel,
        out_type=(jax.ShapeDtypeStruct((EPAD,), jnp.float32),
                  jax.ShapeDtypeStruct((NC * NPAD,), jnp.float32)),
        mesh=_mesh(),
        compiler_params=pltpu.CompilerParams(needs_layout_passes=False),
        scratch_types=[
            pltpu.VMEM((C,), jnp.int32),
            pltpu.VMEM((C,), jnp.int32),
            pltpu.VMEM((C,), jnp.int32),
            pltpu.VMEM((C,), jnp.int32),
            pltpu.VMEM((C, D), jnp.float32),
            pltpu.VMEM((C, D), jnp.float32),
            pltpu.VMEM((C, D), jnp.float32),
            pltpu.VMEM((C, D), jnp.float32),
            pltpu.VMEM((C,), jnp.float32),
            pltpu.VMEM((C,), jnp.float32),
            pltpu.VMEM((NPAD,), jnp.float32),
            pltpu.VMEM((NS, SLICE), jnp.float32),
            pltpu.VMEM((SLICE,), jnp.float32),
            pltpu.VMEM((L,), jnp.int32),
            pltpu.VMEM_SHARED((NS, NPAD), jnp.float32),
            pltpu.SemaphoreType.DMA,
            pltpu.SemaphoreType.DMA,
            pltpu.SemaphoreType.DMA,
            pltpu.SemaphoreType.DMA,
            pltpu.SemaphoreType.DMA,
            pltpu.SemaphoreType.DMA,
        ],
    )
    def kern(q_hbm, k_hbm, src_hbm, dst_hbm, li_hbm, lo_hbm, mp_hbm,
             dstb0, srcb0, dstb1, srcb1, qrows0, krows0, qrows1, krows1,
             lbuf0, lbuf1, mloc, mslice, outsl, lanesb, spm,
             msem0, msem1, gsem0, gsem1, wsem0, wsem1):
        cid = lax.axis_index("c")
        sid = lax.axis_index("s")
        ebase = jnp.where(cid == 0, sid * (CH0 * C),
                          NS * CH0 * C + sid * (CH1 * C))
        nch2 = jnp.where(cid == 0, CH0 // 2, CH1 // 2)

        slots = ((dstb0, srcb0, qrows0, krows0, lbuf0, msem0, gsem0, wsem0),
                 (dstb1, srcb1, qrows1, krows1, lbuf1, msem1, gsem1, wsem1))

        def meta_start(ch, si):
            dstb, srcb, _, _, _, msem, _, _ = slots[si]
            base = ebase + ch * C
            pltpu.async_copy(dst_hbm.at[pl.ds(base, C)], dstb, msem)
            pltpu.async_copy(src_hbm.at[pl.ds(base, C)], srcb, msem)

        def meta_wait(si):
            dstb, srcb, _, _, _, msem, _, _ = slots[si]
            pltpu.make_async_copy(dst_hbm.at[pl.ds(0, C)], dstb, msem).wait()
            pltpu.make_async_copy(src_hbm.at[pl.ds(0, C)], srcb, msem).wait()

        def gather_start(si):
            dstb, srcb, qrows, krows, _, _, gsem, _ = slots[si]
            pltpu.async_copy(q_hbm.at[dstb], qrows, gsem)
            pltpu.async_copy(k_hbm.at[srcb], krows, gsem)

        def gather_wait(si):
            dstb, srcb, qrows, krows, _, _, gsem, _ = slots[si]
            pltpu.make_async_copy(q_hbm.at[dstb], qrows, gsem).wait()
            pltpu.make_async_copy(k_hbm.at[srcb], krows, gsem).wait()

        pltpu.sync_copy(li_hbm, lanesb)
        neg = jnp.full((L,), NEG, jnp.float32)

        def initb(i, _):
            mloc[pl.ds(i * L, L)] = neg
            return 0
        lax.fori_loop(0, NPAD // L, initb, 0)

        lanes = lanesb[...]

        def process(ch, ch2, si):
            dstb, srcb, qrows, krows, lbuf, _, _, wsem = slots[si]
            base = ebase + ch * C

            @pl.when(ch2 > 0)
            def _():
                pltpu.make_async_copy(
                    lbuf, lo_hbm.at[pl.ds(0, C)], wsem).wait()

            def dot_grp(g, _):
                jb = g * L
                lv = jnp.zeros((L,), jnp.float32)
                for jj in range(L):
                    j = jb + jj
                    acc = qrows[j, pl.ds(0, L)] * krows[j, pl.ds(0, L)]
                    for cg in range(1, D // L):
                        acc = acc + (qrows[j, pl.ds(cg * L, L)] *
                                     krows[j, pl.ds(cg * L, L)])
                    s = _lane_sum(acc, lanes) * SCALE
                    lv = jnp.where(lanes == jj, s, lv)
                lbuf[pl.ds(jb, L)] = lv
                return 0
            lax.fori_loop(0, C // L, dot_grp, 0)

            pltpu.async_copy(lbuf, lo_hbm.at[pl.ds(base, C)], wsem)

            def grp(g, _):
                dv = dstb[pl.ds(g * L, L)]
                lv = lbuf[pl.ds(g * L, L)]
                acc = lv
                for r in range(1, L):
                    perm = (lanes + r) & (L - 1)
                    ki = jnp.take_along_axis(dv, perm, axis=0)
                    vi = jnp.take_along_axis(lv, perm, axis=0)
                    acc = jnp.where(dv == ki, jnp.maximum(acc, vi), acc)
                cur = plsc.load_gather(mloc, [dv])
                plsc.store_scatter(mloc, [dv], jnp.maximum(cur, acc))
                return 0
            lax.fori_loop(0, C // L, grp, 0)

        meta_start(0, 0)
        meta_wait(0)
        gather_start(0)
        meta_start(1, 1)

        def body2(ch2, _):
            c0 = 2 * ch2
            meta_wait(1)
            gather_start(1)
            gather_wait(0)
            process(c0, ch2, 0)

            @pl.when(ch2 + 1 < nch2)
            def _():
                meta_start(c0 + 2, 0)
                pltpu.make_async_copy(
                    dst_hbm.at[pl.ds(0, C)], dstb0, msem0).wait()
                pltpu.make_async_copy(
                    src_hbm.at[pl.ds(0, C)], srcb0, msem0).wait()
                gather_start(0)

            gather_wait(1)
            process(c0 + 1, ch2, 1)

            @pl.when(ch2 + 1 < nch2)
            def _():
                meta_start(c0 + 3, 1)
            return 0
        lax.fori_loop(0, nch2, body2, 0)
        pltpu.make_async_copy(lbuf0, lo_hbm.at[pl.ds(0, C)], wsem0).wait()
        pltpu.make_async_copy(lbuf1, lo_hbm.at[pl.ds(0, C)], wsem1).wait()

        pltpu.sync_copy(mloc, spm.at[sid])
        plsc.subcore_barrier()
        off = sid * SLICE
        for r in range(NS):
            pltpu.sync_copy(spm.at[r, pl.ds(off, SLICE)], mslice.at[r])

        def red(gi, _):
            a = mslice[0, pl.ds(gi * L, L)]
            for r in range(1, NS):
                a = jnp.maximum(a, mslice[r, pl.ds(gi * L, L)])
            outsl[pl.ds(gi * L, L)] = a
            return 0
        lax.fori_loop(0, SLICE // L, red, 0)
        pltpu.sync_copy(outsl, mp_hbm.at[pl.ds(cid * NPAD + off, SLICE)])

    return kern(q, k, srcp, dstp, lane_iota)


NACC = 10112
ASL = NACC // NS
CH0 = 56
CH1 = 2 * NCH - CH0


def _sc_softmax_agg(v, srcp, dstp, lo, mp):
    @functools.partial(
        pl.kernel,
        out_type=(jax.ShapeDtypeStruct((NW * NPAD,), jnp.float32),
                  jax.ShapeDtypeStruct((NC, NACC, D), jnp.float32)),
        mesh=_mesh(),
        compiler_params=pltpu.CompilerParams(needs_layout_passes=False),
        scratch_types=[
            pltpu.VMEM((2, C), jnp.int32),
            pltpu.VMEM((2, C), jnp.int32),
            pltpu.VMEM((C, D), jnp.float32),
            pltpu.VMEM((2, C), jnp.float32),
            pltpu.VMEM((2, C), jnp.float32),
            pltpu.VMEM((NPAD,), jnp.float32),
            pltpu.VMEM((NPAD,), jnp.float32),
            pltpu.VMEM((NPAD,), jnp.float32),
            pltpu.VMEM_SHARED((NACC, D), jnp.float32),
            pltpu.SemaphoreType.DMA,
            pltpu.SemaphoreType.DMA,
            pltpu.SemaphoreType.DMA,
            pltpu.SemaphoreType.DMA,
            pltpu.SemaphoreType.DMA,
            pltpu.SemaphoreType.DMA,
        ],
    )
    def kern(v_hbm, src_hbm, dst_hbm, lo_hbm, mp_hbm, dn_hbm, agg_hbm,
             dstb2, srcb2, vrows0, lbuf2, ebuf2,
             m0, m1, dloc, agg_spm,
             msem0, msem1, gsem0, gsem1, ssem0, ssem1):
        cid = lax.axis_index("c")
        sid = lax.axis_index("s")
        wid = cid * NS + sid
        ebase = jnp.where(cid == 0, sid * (CH0 * C),
                          NS * CH0 * C + sid * (CH1 * C))
        nch2 = jnp.where(cid == 0, CH0 // 2, CH1 // 2)

        slots = ((dstb2.at[0], srcb2.at[0], vrows0,
                  lbuf2.at[0], ebuf2.at[0], msem0, gsem0, ssem0),
                 (dstb2.at[1], srcb2.at[1], vrows0,
                  lbuf2.at[1], ebuf2.at[1], msem1, gsem1, ssem1))

        def meta_start(ch, si):
            dstb, srcb, _, lbuf, _, msem, _, _ = slots[si]
            base = ebase + ch * C
            pltpu.async_copy(dst_hbm.at[pl.ds(base, C)], dstb, msem)
            pltpu.async_copy(src_hbm.at[pl.ds(base, C)], srcb, msem)
            pltpu.async_copy(lo_hbm.at[pl.ds(base, C)], lbuf, msem)

        def meta_wait(si):
            dstb, srcb, _, lbuf, _, msem, _, _ = slots[si]
            pltpu.make_async_copy(dst_hbm.at[pl.ds(0, C)], dstb, msem).wait()
            pltpu.make_async_copy(src_hbm.at[pl.ds(0, C)], srcb, msem).wait()
            pltpu.make_async_copy(lo_hbm.at[pl.ds(0, C)], lbuf, msem).wait()

        def gather_start(si):
            _, srcb, vrows, _, _, _, gsem, _ = slots[si]
            pltpu.async_copy(v_hbm.at[srcb], vrows, gsem)

        def gather_wait(si):
            _, srcb, vrows, _, _, _, gsem, _ = slots[si]
            pltpu.make_async_copy(v_hbm.at[srcb], vrows, gsem).wait()

        def scatter_start(si):
            dstb, _, vrows, _, _, _, _, ssem = slots[si]
            pltpu.async_copy(vrows, agg_spm.at[dstb], ssem, add=True)

        def scatter_wait(si):
            dstb, _, vrows, _, _, _, _, ssem = slots[si]
            pltpu.make_async_copy(vrows, agg_spm.at[dstb], ssem).wait()

        pltpu.sync_copy(mp_hbm.at[pl.ds(0, NPAD)], m0)
        pltpu.sync_copy(mp_hbm.at[pl.ds(NPAD, NPAD)], m1)

        zer = jnp.zeros((L,), jnp.float32)

        def initd(i, _):
            dloc[pl.ds(i * L, L)] = zer
            return 0
        lax.fori_loop(0, NPAD // L, initd, 0)

        def initv(j, _):
            for cg in range(D // L):
                vrows0[j, pl.ds(cg * L, L)] = zer
            return 0
        lax.fori_loop(0, C, initv, 0)
        rb = sid * ASL
        for t in range(ASL // C):
            pltpu.sync_copy(vrows0, agg_spm.at[pl.ds(rb + t * C, C)])
        pltpu.sync_copy(vrows0.at[pl.ds(0, ASL % C)],
                        agg_spm.at[pl.ds(rb + (ASL // C) * C, ASL % C)])
        plsc.subcore_barrier()

        def grp_phase(si):
            dstb, _, _, lbuf, ebuf, _, _, _ = slots[si]

            def grp(g, _):
                dv = dstb[pl.ds(g * L, L)]
                lv = lbuf[pl.ds(g * L, L)]
                ma = jnp.maximum(plsc.load_gather(m0, [dv]),
                                 plsc.load_gather(m1, [dv]))
                ev = jnp.exp(lv - ma)
                ebuf[pl.ds(g * L, L)] = ev
                plsc.addupdate_scatter(dloc, [dv], ev)
                return 0
            lax.fori_loop(0, C // L, grp, 0)

        def srow_phase(si):
            _, _, vrows, _, ebuf, _, _, _ = slots[si]

            def srow_grp(g, _):
                jb = g * L
                ev = ebuf[pl.ds(jb, L)]
                for jj in range(L):
                    evs = ev[jj]
                    j = jb + jj
                    for cg in range(D // L):
                        vrows[j, pl.ds(cg * L, L)] = (
                            vrows[j, pl.ds(cg * L, L)] * evs)
                return 0
            lax.fori_loop(0, C // L, srow_grp, 0)

        meta_start(0, 0)
        meta_start(1, 1)

        def body2(ch2, _):
            c0 = 2 * ch2
            meta_wait(0)
            grp_phase(0)
            gather_start(0)
            gather_wait(0)
            srow_phase(0)
            scatter_start(0)

            meta_wait(1)
            grp_phase(1)

            @pl.when(ch2 + 1 < nch2)
            def _():
                meta_start(c0 + 2, 0)
            scatter_wait(0)
            gather_start(1)
            gather_wait(1)
            srow_phase(1)
            scatter_start(1)

            @pl.when(ch2 + 1 < nch2)
            def _():
                meta_start(c0 + 3, 1)
            scatter_wait(1)
            return 0
        lax.fori_loop(0, nch2, body2, 0)

        plsc.subcore_barrier()
        pltpu.sync_copy(agg_spm.at[pl.ds(rb, ASL)],
                        agg_hbm.at[cid, pl.ds(rb, ASL)])

        pltpu.sync_copy(dloc, dn_hbm.at[pl.ds(wid * NPAD, NPAD)])

    return kern(v, srcp, dstp, lo, mp)


def kernel(x, edge_index, Wq1, bq1, Wk1, bk1, Wv1, bv1, Ws1, bs1,
           Wq2, bq2, Wk2, bk2, Wv2, bv2, Ws2, bs2,
           Wl1, bl1, Wl2, bl2, Wr1, br1, Wr2, br2):
    x = jax.lax.stop_gradient(x)
    src = edge_index[0].astype(jnp.int32)
    dst = edge_index[1].astype(jnp.int32)
    pad_idx = jnp.full((EPAD - E,), N, jnp.int32)
    srcp = jnp.concatenate([src, pad_idx])
    dstp = jnp.concatenate([dst, pad_idx])
    xp = jnp.pad(x, ((0, NPAD - N), (0, 0)))
    lane_iota = jnp.arange(L, dtype=jnp.int32)

    q1, k1, v1, s1 = _proj4_call(xp, Wq1, bq1, Wk1, bk1, Wv1, bv1, Ws1, bs1)
    lo1, mp1 = _sc_logits_max(q1, k1, srcp, dstp, lane_iota)
    dp1, ap1 = _sc_softmax_agg(v1, srcp, dstp, lo1, mp1)

    def pad_acc(a):
        return jnp.pad(a, ((0, 0), (0, NPAD - NACC), (0, 0)))

    ap1p = pad_acc(ap1)
    q2, k2, v2, s2 = _proj4_call(
        s1, Wq2, bq2, Wk2, bk2, Wv2, bv2, Ws2, bs2, relu_in=True,
        a0=ap1p[0], a1=ap1p[1], den=dp1.reshape(NW, NPAD).T)
    lo2, mp2 = _sc_logits_max(q2, k2, srcp, dstp, lane_iota)
    dp2, ap2 = _sc_softmax_agg(v2, srcp, dstp, lo2, mp2)

    ap2p = pad_acc(ap2)
    s_vec = _finalize_call(
        ap2p[0], ap2p[1], dp2.reshape(NW, NPAD).T,
        s2, Wl1, bl1, Wl2, bl2, Wr1, br1, Wr2, br2)

    s_n = s_vec[:N]
    return _outer_call(s_n, s_n.reshape(1, N))


# speedup vs baseline: 1.2249x; 1.2249x over previous
import functools
import math

import jax
import jax.numpy as jnp
from jax import lax
from jax.experimental import pallas as pl
from jax.experimental.pallas import tpu as pltpu
from jax.experimental.pallas import tpu_sc as plsc

N = 10000
E = 320000
D = 128
NC = 2
NS = 16
L = 16
NW = NC * NS
NPAD = 10240
EPAD = 327680
EPW = EPAD // NW
C = 128
NCH = EPW // C
SLICE = NPAD // NS
NEG = -3.0e38
SCALE = 1.0 / math.sqrt(float(D))

_mesh = functools.partial(
    plsc.VectorSubcoreMesh,
    core_axis_name="c", subcore_axis_name="s", num_cores=NC, num_subcores=NS)


def _lane_sum(a, lanes):
    for k in (8, 4, 2, 1):
        perm = (lanes + k) & (L - 1)
        a = a + jnp.take_along_axis(a, perm, axis=0)
    return a


def _proj4_call(h, Wq, bq, Wk, bk, Wv, bv, Ws, bs, relu_in=False,
                a0=None, a1=None, den=None):
    BP = 1024
    grid = (NPAD // BP,)
    row_spec = pl.BlockSpec((BP, D), lambda i: (i, 0))
    w_spec = pl.BlockSpec((D, D), lambda i: (0, 0))
    b_spec = pl.BlockSpec((1, D), lambda i: (0, 0))
    den_spec = pl.BlockSpec((BP, NW), lambda i: (i, 0))

    combine = a0 is not None

    def body(*refs):
        if combine:
            a0r, a1r, dr, skr, wqr, bqr, wkr, bkr, wvr, bvr, wsr, bsr, \
                qo, ko, vo, so = refs
            dd = jnp.sum(dr[...], axis=1, keepdims=True) + 1e-16
            hb = (a0r[...] + a1r[...]) / dd + skr[...]
            if relu_in:
                hb = jnp.maximum(hb, 0.0)
        else:
            hr, wqr, bqr, wkr, bkr, wvr, bvr, wsr, bsr, qo, ko, vo, so = refs
            hb = hr[...]
        qo[...] = jnp.dot(hb, wqr[...], preferred_element_type=jnp.float32) + bqr[...]
        ko[...] = jnp.dot(hb, wkr[...], preferred_element_type=jnp.float32) + bkr[...]
        vo[...] = jnp.dot(hb, wvr[...], preferred_element_type=jnp.float32) + bvr[...]
        so[...] = jnp.dot(hb, wsr[...], preferred_element_type=jnp.float32) + bsr[...]

    if combine:
        in_specs = [row_spec, row_spec, den_spec, row_spec] + \
                   [w_spec, b_spec] * 4
        args = (a0, a1, den, h,
                Wq, bq.reshape(1, D), Wk, bk.reshape(1, D),
                Wv, bv.reshape(1, D), Ws, bs.reshape(1, D))
    else:
        in_specs = [row_spec] + [w_spec, b_spec] * 4
        args = (h, Wq, bq.reshape(1, D), Wk, bk.reshape(1, D),
                Wv, bv.reshape(1, D), Ws, bs.reshape(1, D))

    out = jax.ShapeDtypeStruct((NPAD, D), jnp.float32)
    return pl.pallas_call(
        body, grid=grid, in_specs=in_specs,
        out_specs=[row_spec] * 4, out_shape=[out] * 4,
    )(*args)


def _finalize_call(a0, a1, den, sk, Wl1, bl1, Wl2, bl2, Wr1, br1, Wr2, br2):
    BP = 1024
    grid = (NPAD // BP,)
    row_spec = pl.BlockSpec((BP, D), lambda i: (i, 0))
    w_spec = pl.BlockSpec((D, D), lambda i: (0, 0))
    b_spec = pl.BlockSpec((1, D), lambda i: (0, 0))
    w2_spec = pl.BlockSpec((D, 1), lambda i: (0, 0))
    b2_spec = pl.BlockSpec((1, 1), lambda i: (0, 0))
    col_spec = pl.BlockSpec((BP, 1), lambda i: (i, 0))

    den_spec = pl.BlockSpec((BP, NW), lambda i: (i, 0))

    def body(a0r, a1r, dr, skr, wl1, bl1r, wl2, bl2r,
             wr1, br1r, wr2, br2r, so):
        dd = jnp.sum(dr[...], axis=1, keepdims=True) + 1e-16
        f = (a0r[...] + a1r[...]) / dd + skr[...]
        tl = jnp.dot(f, wl1[...], preferred_element_type=jnp.float32) + bl1r[...]
        al = jnp.dot(tl, wl2[...], preferred_element_type=jnp.float32) + bl2r[...]
        tr = jnp.dot(f, wr1[...], preferred_element_type=jnp.float32) + br1r[...]
        ar = jnp.dot(tr, wr2[...], preferred_element_type=jnp.float32) + br2r[...]
        so[...] = (al + ar) * 0.5

    return pl.pallas_call(
        body, grid=grid,
        in_specs=[row_spec, row_spec, den_spec, row_spec,
                  w_spec, b_spec, w2_spec, b2_spec,
                  w_spec, b_spec, w2_spec, b2_spec],
        out_specs=col_spec,
        out_shape=jax.ShapeDtypeStruct((NPAD, 1), jnp.float32),
    )(a0, a1, den, sk,
      Wl1, bl1.reshape(1, D), Wl2, bl2.reshape(1, 1),
      Wr1, br1.reshape(1, D), Wr2, br2.reshape(1, 1))


def _outer_call(s_row, s_col):
    BR = 256
    grid = (pl.cdiv(N, BR),)

    def body(sr, sc, o):
        z = sr[...] + sc[...]
        o[...] = 1.0 / (1.0 + jnp.exp(-z))

    return pl.pallas_call(
        body, grid=grid,
        in_specs=[pl.BlockSpec((BR, 1), lambda i: (i, 0)),
                  pl.BlockSpec((1, N), lambda i: (0, 0))],
        out_specs=pl.BlockSpec((BR, N), lambda i: (i, 0)),
        out_shape=jax.ShapeDtypeStruct((N, N), jnp.float32),
    )(s_row, s_col)


def _sc_logits_max(q, k, srcp, dstp, lane_iota):
    @functools.partial(
        pl.kernel,
        out_type=(jax.ShapeDtypeStruct((EPAD,), jnp.float32),
                  jax.ShapeDtypeStruct((NC * NPAD,), jnp.float32)),
        mesh=_mesh(),
        compiler_params=pltpu.CompilerParams(needs_layout_passes=False),
        scratch_types=[
            pltpu.VMEM((C,), jnp.int32),
            pltpu.VMEM((C,), jnp.int32),
            pltpu.VMEM((C,), jnp.int32),
            pltpu.VMEM((C,), jnp.int32),
            pltpu.VMEM((C, D), jnp.float32),
            pltpu.VMEM((C, D), jnp.float32),
            pltpu.VMEM((C, D), jnp.float32),
            pltpu.VMEM((C, D), jnp.float32),
            pltpu.VMEM((C,), jnp.float32),
            pltpu.VMEM((C,), jnp.float32),
            pltpu.VMEM((NPAD,), jnp.float32),
            pltpu.VMEM((NS, SLICE), jnp.float32),
            pltpu.VMEM((SLICE,), jnp.float32),
            pltpu.VMEM((L,), jnp.int32),
            pltpu.VMEM_SHARED((NS, NPAD), jnp.float32),
            pltpu.SemaphoreType.DMA,
            pltpu.SemaphoreType.DMA,
            pltpu.SemaphoreType.DMA,
            pltpu.SemaphoreType.DMA,
            pltpu.SemaphoreType.DMA,
            pltpu.SemaphoreType.DMA,
        ],
    )
    def kern(q_hbm, k_hbm, src_hbm, dst_hbm, li_hbm, lo_hbm, mp_hbm,
             dstb0, srcb0, dstb1, srcb1, qrows0, krows0, qrows1, krows1,
             lbuf0, lbuf1, mloc, mslice, outsl, lanesb, spm,
             msem0, msem1, gsem0, gsem1, wsem0, wsem1):
        cid = lax.axis_index("c")
        sid = lax.axis_index("s")
        ebase = jnp.where(cid == 0, sid * (CH0 * C),
                          NS * CH0 * C + sid * (CH1 * C))
        nch2 = jnp.where(cid == 0, CH0 // 2, CH1 // 2)

        slots = ((dstb0, srcb0, qrows0, krows0, lbuf0, msem0, gsem0, wsem0),
                 (dstb1, srcb1, qrows1, krows1, lbuf1, msem1, gsem1, wsem1))

        def meta_start(ch, si):
            dstb, srcb, _, _, _, msem, _, _ = slots[si]
            base = ebase + ch * C
            pltpu.async_copy(dst_hbm.at[pl.ds(base, C)], dstb, msem)
            pltpu.async_copy(src_hbm.at[pl.ds(base, C)], srcb, msem)

        def meta_wait(si):
            dstb, srcb, _, _, _, msem, _, _ = slots[si]
            pltpu.make_async_copy(dst_hbm.at[pl.ds(0, C)], dstb, msem).wait()
            pltpu.make_async_copy(src_hbm.at[pl.ds(0, C)], srcb, msem).wait()

        def gather_start(si):
            dstb, srcb, qrows, krows, _, _, gsem, _ = slots[si]
            pltpu.async_copy(q_hbm.at[dstb], qrows, gsem)
            pltpu.async_copy(k_hbm.at[srcb], krows, gsem)

        def gather_wait(si):
            dstb, srcb, qrows, krows, _, _, gsem, _ = slots[si]
            pltpu.make_async_copy(q_hbm.at[dstb], qrows, gsem).wait()
            pltpu.make_async_copy(k_hbm.at[srcb], krows, gsem).wait()

        pltpu.sync_copy(li_hbm, lanesb)
        neg = jnp.full((L,), NEG, jnp.float32)

        def initb(i, _):
            mloc[pl.ds(i * L, L)] = neg
            return 0
        lax.fori_loop(0, NPAD // L, initb, 0)

        lanes = lanesb[...]

        def process(ch, ch2, si):
            dstb, srcb, qrows, krows, lbuf, _, _, wsem = slots[si]
            base = ebase + ch * C

            @pl.when(ch2 > 0)
            def _():
                pltpu.make_async_copy(
                    lbuf, lo_hbm.at[pl.ds(0, C)], wsem).wait()

            def dot_grp(g, _):
                jb = g * L
                lv = jnp.zeros((L,), jnp.float32)
                for jj in range(L):
                    j = jb + jj
                    acc = qrows[j, pl.ds(0, L)] * krows[j, pl.ds(0, L)]
                    for cg in range(1, D // L):
                        acc = acc + (qrows[j, pl.ds(cg * L, L)] *
                                     krows[j, pl.ds(cg * L, L)])
                    s = _lane_sum(acc, lanes) * SCALE
                    lv = jnp.where(lanes == jj, s, lv)
                lbuf[pl.ds(jb, L)] = lv
                return 0
            lax.fori_loop(0, C // L, dot_grp, 0)

            pltpu.async_copy(lbuf, lo_hbm.at[pl.ds(base, C)], wsem)

            def grp(g, _):
                dv = dstb[pl.ds(g * L, L)]
                lv = lbuf[pl.ds(g * L, L)]
                acc = lv
                for r in range(1, L):
                    perm = (lanes + r) & (L - 1)
                    ki = jnp.take_along_axis(dv, perm, axis=0)
                    vi = jnp.take_along_axis(lv, perm, axis=0)
                    acc = jnp.where(dv == ki, jnp.maximum(acc, vi), acc)
                cur = plsc.load_gather(mloc, [dv])
                plsc.store_scatter(mloc, [dv], jnp.maximum(cur, acc))
                return 0
            lax.fori_loop(0, C // L, grp, 0)

        meta_start(0, 0)
        meta_wait(0)
        gather_start(0)
        meta_start(1, 1)

        def body2(ch2, _):
            c0 = 2 * ch2
            meta_wait(1)
            gather_start(1)
            gather_wait(0)
            process(c0, ch2, 0)

            @pl.when(ch2 + 1 < nch2)
            def _():
                meta_start(c0 + 2, 0)
                pltpu.make_async_copy(
                    dst_hbm.at[pl.ds(0, C)], dstb0, msem0).wait()
                pltpu.make_async_copy(
                    src_hbm.at[pl.ds(0, C)], srcb0, msem0).wait()
                gather_start(0)

            gather_wait(1)
            process(c0 + 1, ch2, 1)

            @pl.when(ch2 + 1 < nch2)
            def _():
                meta_start(c0 + 3, 1)
            return 0
        lax.fori_loop(0, nch2, body2, 0)
        pltpu.make_async_copy(lbuf0, lo_hbm.at[pl.ds(0, C)], wsem0).wait()
        pltpu.make_async_copy(lbuf1, lo_hbm.at[pl.ds(0, C)], wsem1).wait()

        pltpu.sync_copy(mloc, spm.at[sid])
        plsc.subcore_barrier()
        off = sid * SLICE
        for r in range(NS):
            pltpu.sync_copy(spm.at[r, pl.ds(off, SLICE)], mslice.at[r])

        def red(gi, _):
            a = mslice[0, pl.ds(gi * L, L)]
            for r in range(1, NS):
                a = jnp.maximum(a, mslice[r, pl.ds(gi * L, L)])
            outsl[pl.ds(gi * L, L)] = a
            return 0
        lax.fori_loop(0, SLICE // L, red, 0)
        pltpu.sync_copy(outsl, mp_hbm.at[pl.ds(cid * NPAD + off, SLICE)])

    return kern(q, k, srcp, dstp, lane_iota)


NACC = 10112
ASL = NACC // NS
CH0 = 104
CH1 = 2 * NCH - CH0


def _sc_softmax_agg(v, srcp, dstp, lo, mp):
    @functools.partial(
        pl.kernel,
        out_type=(jax.ShapeDtypeStruct((NW * NPAD,), jnp.float32),
                  jax.ShapeDtypeStruct((NC, NACC, D), jnp.float32)),
        mesh=_mesh(),
        compiler_params=pltpu.CompilerParams(needs_layout_passes=False),
        scratch_types=[
            pltpu.VMEM((2, C), jnp.int32),
            pltpu.VMEM((2, C), jnp.int32),
            pltpu.VMEM((C, D), jnp.float32),
            pltpu.VMEM((2, C), jnp.float32),
            pltpu.VMEM((2, C), jnp.float32),
            pltpu.VMEM((NPAD,), jnp.float32),
            pltpu.VMEM((NPAD,), jnp.float32),
            pltpu.VMEM((NPAD,), jnp.float32),
            pltpu.VMEM_SHARED((NACC, D), jnp.float32),
            pltpu.SemaphoreType.DMA,
            pltpu.SemaphoreType.DMA,
            pltpu.SemaphoreType.DMA,
            pltpu.SemaphoreType.DMA,
            pltpu.SemaphoreType.DMA,
            pltpu.SemaphoreType.DMA,
        ],
    )
    def kern(v_hbm, src_hbm, dst_hbm, lo_hbm, mp_hbm, dn_hbm, agg_hbm,
             dstb2, srcb2, vrows0, lbuf2, ebuf2,
             m0, m1, dloc, agg_spm,
             msem0, msem1, gsem0, gsem1, ssem0, ssem1):
        cid = lax.axis_index("c")
        sid = lax.axis_index("s")
        wid = cid * NS + sid
        ebase = jnp.where(cid == 0, sid * (CH0 * C),
                          NS * CH0 * C + sid * (CH1 * C))
        nch2 = jnp.where(cid == 0, CH0 // 2, CH1 // 2)

        slots = ((dstb2.at[0], srcb2.at[0], vrows0,
                  lbuf2.at[0], ebuf2.at[0], msem0, gsem0, ssem0),
                 (dstb2.at[1], srcb2.at[1], vrows0,
                  lbuf2.at[1], ebuf2.at[1], msem1, gsem1, ssem1))

        def meta_start(ch, si):
            dstb, srcb, _, lbuf, _, msem, _, _ = slots[si]
            base = ebase + ch * C
            pltpu.async_copy(dst_hbm.at[pl.ds(base, C)], dstb, msem)
            pltpu.async_copy(src_hbm.at[pl.ds(base, C)], srcb, msem)
            pltpu.async_copy(lo_hbm.at[pl.ds(base, C)], lbuf, msem)

        def meta_wait(si):
            dstb, srcb, _, lbuf, _, msem, _, _ = slots[si]
            pltpu.make_async_copy(dst_hbm.at[pl.ds(0, C)], dstb, msem).wait()
            pltpu.make_async_copy(src_hbm.at[pl.ds(0, C)], srcb, msem).wait()
            pltpu.make_async_copy(lo_hbm.at[pl.ds(0, C)], lbuf, msem).wait()

        def gather_start(si):
            _, srcb, vrows, _, _, _, gsem, _ = slots[si]
            pltpu.async_copy(v_hbm.at[srcb], vrows, gsem)

        def gather_wait(si):
            _, srcb, vrows, _, _, _, gsem, _ = slots[si]
            pltpu.make_async_copy(v_hbm.at[srcb], vrows, gsem).wait()

        def scatter_start(si):
            dstb, _, vrows, _, _, _, _, ssem = slots[si]
            pltpu.async_copy(vrows, agg_spm.at[dstb], ssem, add=True)

        def scatter_wait(si):
            dstb, _, vrows, _, _, _, _, ssem = slots[si]
            pltpu.make_async_copy(vrows, agg_spm.at[dstb], ssem).wait()

        pltpu.sync_copy(mp_hbm.at[pl.ds(0, NPAD)], m0)
        pltpu.sync_copy(mp_hbm.at[pl.ds(NPAD, NPAD)], m1)

        zer = jnp.zeros((L,), jnp.float32)

        def initd(i, _):
            dloc[pl.ds(i * L, L)] = zer
            return 0
        lax.fori_loop(0, NPAD // L, initd, 0)

        def initv(j, _):
            for cg in range(D // L):
                vrows0[j, pl.ds(cg * L, L)] = zer
            return 0
        lax.fori_loop(0, C, initv, 0)
        rb = sid * ASL
        for t in range(ASL // C):
            pltpu.sync_copy(vrows0, agg_spm.at[pl.ds(rb + t * C, C)])
        pltpu.sync_copy(vrows0.at[pl.ds(0, ASL % C)],
                        agg_spm.at[pl.ds(rb + (ASL // C) * C, ASL % C)])
        plsc.subcore_barrier()

        def grp_phase(si):
            dstb, _, _, lbuf, ebuf, _, _, _ = slots[si]

            def grp(g, _):
                dv = dstb[pl.ds(g * L, L)]
                lv = lbuf[pl.ds(g * L, L)]
                ma = jnp.maximum(plsc.load_gather(m0, [dv]),
                                 plsc.load_gather(m1, [dv]))
                ev = jnp.exp(lv - ma)
                ebuf[pl.ds(g * L, L)] = ev
                plsc.addupdate_scatter(dloc, [dv], ev)
                return 0
            lax.fori_loop(0, C // L, grp, 0)

        def srow_phase(si):
            _, _, vrows, _, ebuf, _, _, _ = slots[si]

            def srow_grp(g, _):
                jb = g * L
                ev = ebuf[pl.ds(jb, L)]
                for jj in range(L):
                    evs = ev[jj]
                    j = jb + jj
                    for cg in range(D // L):
                        vrows[j, pl.ds(cg * L, L)] = (
                            vrows[j, pl.ds(cg * L, L)] * evs)
                return 0
            lax.fori_loop(0, C // L, srow_grp, 0)

        meta_start(0, 0)
        meta_start(1, 1)

        def body2(ch2, _):
            c0 = 2 * ch2
            meta_wait(0)
            grp_phase(0)
            gather_start(0)
            gather_wait(0)
            srow_phase(0)
            scatter_start(0)

            meta_wait(1)
            grp_phase(1)

            @pl.when(ch2 + 1 < nch2)
            def _():
                meta_start(c0 + 2, 0)
            scatter_wait(0)
            gather_start(1)
            gather_wait(1)
            srow_phase(1)
            scatter_start(1)

            @pl.when(ch2 + 1 < nch2)
            def _():
                meta_start(c0 + 3, 1)
            scatter_wait(1)
            return 0
        lax.fori_loop(0, nch2, body2, 0)

        plsc.subcore_barrier()
        pltpu.sync_copy(agg_spm.at[pl.ds(rb, ASL)],
                        agg_hbm.at[cid, pl.ds(rb, ASL)])

        pltpu.sync_copy(dloc, dn_hbm.at[pl.ds(wid * NPAD, NPAD)])

    return kern(v, srcp, dstp, lo, mp)


def kernel(x, edge_index, Wq1, bq1, Wk1, bk1, Wv1, bv1, Ws1, bs1,
           Wq2, bq2, Wk2, bk2, Wv2, bv2, Ws2, bs2,
           Wl1, bl1, Wl2, bl2, Wr1, br1, Wr2, br2):
    x = jax.lax.stop_gradient(x)
    src = edge_index[0].astype(jnp.int32)
    dst = edge_index[1].astype(jnp.int32)
    pad_idx = jnp.full((EPAD - E,), N, jnp.int32)
    srcp = jnp.concatenate([src, pad_idx])
    dstp = jnp.concatenate([dst, pad_idx])
    xp = jnp.pad(x, ((0, NPAD - N), (0, 0)))
    lane_iota = jnp.arange(L, dtype=jnp.int32)

    q1, k1, v1, s1 = _proj4_call(xp, Wq1, bq1, Wk1, bk1, Wv1, bv1, Ws1, bs1)
    lo1, mp1 = _sc_logits_max(q1, k1, srcp, dstp, lane_iota)
    dp1, ap1 = _sc_softmax_agg(v1, srcp, dstp, lo1, mp1)

    def pad_acc(a):
        return jnp.pad(a, ((0, 0), (0, NPAD - NACC), (0, 0)))

    ap1p = pad_acc(ap1)
    q2, k2, v2, s2 = _proj4_call(
        s1, Wq2, bq2, Wk2, bk2, Wv2, bv2, Ws2, bs2, relu_in=True,
        a0=ap1p[0], a1=ap1p[1], den=dp1.reshape(NW, NPAD).T)
    lo2, mp2 = _sc_logits_max(q2, k2, srcp, dstp, lane_iota)
    dp2, ap2 = _sc_softmax_agg(v2, srcp, dstp, lo2, mp2)

    ap2p = pad_acc(ap2)
    s_vec = _finalize_call(
        ap2p[0], ap2p[1], dp2.reshape(NW, NPAD).T,
        s2, Wl1, bl1, Wl2, bl2, Wr1, br1, Wr2, br2)

    s_n = s_vec[:N]
    return _outer_call(s_n, s_n.reshape(1, N))


# speedup vs baseline: 1.2550x; 1.0246x over previous
import functools
import math

import jax
import jax.numpy as jnp
from jax import lax
from jax.experimental import pallas as pl
from jax.experimental.pallas import tpu as pltpu
from jax.experimental.pallas import tpu_sc as plsc

N = 10000
E = 320000
D = 128
NC = 2
NS = 16
L = 16
NW = NC * NS
NPAD = 10240
EPAD = 327680
EPW = EPAD // NW
C = 128
NCH = EPW // C
SLICE = NPAD // NS
NEG = -3.0e38
SCALE = 1.0 / math.sqrt(float(D))

_mesh = functools.partial(
    plsc.VectorSubcoreMesh,
    core_axis_name="c", subcore_axis_name="s", num_cores=NC, num_subcores=NS)


def _lane_sum(a, lanes):
    for k in (8, 4, 2, 1):
        perm = (lanes + k) & (L - 1)
        a = a + jnp.take_along_axis(a, perm, axis=0)
    return a


def _proj4_call(h, Wq, bq, Wk, bk, Wv, bv, Ws, bs, relu_in=False,
                a0=None, a1=None, den=None):
    BP = 1024
    grid = (NPAD // BP,)
    row_spec = pl.BlockSpec((BP, D), lambda i: (i, 0))
    w_spec = pl.BlockSpec((D, D), lambda i: (0, 0))
    b_spec = pl.BlockSpec((1, D), lambda i: (0, 0))
    den_spec = pl.BlockSpec((BP, NW), lambda i: (i, 0))

    combine = a0 is not None

    def body(*refs):
        if combine:
            a0r, a1r, dr, skr, wqr, bqr, wkr, bkr, wvr, bvr, wsr, bsr, \
                qo, ko, vo, so = refs
            dd = jnp.sum(dr[...], axis=1, keepdims=True) + 1e-16
            hb = (a0r[...] + a1r[...]) / dd + skr[...]
            if relu_in:
                hb = jnp.maximum(hb, 0.0)
        else:
            hr, wqr, bqr, wkr, bkr, wvr, bvr, wsr, bsr, qo, ko, vo, so = refs
            hb = hr[...]
        qo[...] = jnp.dot(hb, wqr[...], preferred_element_type=jnp.float32) + bqr[...]
        ko[...] = jnp.dot(hb, wkr[...], preferred_element_type=jnp.float32) + bkr[...]
        vo[...] = jnp.dot(hb, wvr[...], preferred_element_type=jnp.float32) + bvr[...]
        so[...] = jnp.dot(hb, wsr[...], preferred_element_type=jnp.float32) + bsr[...]

    if combine:
        in_specs = [row_spec, row_spec, den_spec, row_spec] + \
                   [w_spec, b_spec] * 4
        args = (a0, a1, den, h,
                Wq, bq.reshape(1, D), Wk, bk.reshape(1, D),
                Wv, bv.reshape(1, D), Ws, bs.reshape(1, D))
    else:
        in_specs = [row_spec] + [w_spec, b_spec] * 4
        args = (h, Wq, bq.reshape(1, D), Wk, bk.reshape(1, D),
                Wv, bv.reshape(1, D), Ws, bs.reshape(1, D))

    out = jax.ShapeDtypeStruct((NPAD, D), jnp.float32)
    return pl.pallas_call(
        body, grid=grid, in_specs=in_specs,
        out_specs=[row_spec] * 4, out_shape=[out] * 4,
    )(*args)


def _finalize_call(a0, a1, den, sk, Wl1, bl1, Wl2, bl2, Wr1, br1, Wr2, br2):
    BP = 1024
    grid = (NPAD // BP,)
    row_spec = pl.BlockSpec((BP, D), lambda i: (i, 0))
    w_spec = pl.BlockSpec((D, D), lambda i: (0, 0))
    b_spec = pl.BlockSpec((1, D), lambda i: (0, 0))
    w2_spec = pl.BlockSpec((D, 1), lambda i: (0, 0))
    b2_spec = pl.BlockSpec((1, 1), lambda i: (0, 0))
    col_spec = pl.BlockSpec((BP, 1), lambda i: (i, 0))

    den_spec = pl.BlockSpec((BP, NW), lambda i: (i, 0))

    def body(a0r, a1r, dr, skr, wl1, bl1r, wl2, bl2r,
             wr1, br1r, wr2, br2r, so):
        dd = jnp.sum(dr[...], axis=1, keepdims=True) + 1e-16
        f = (a0r[...] + a1r[...]) / dd + skr[...]
        tl = jnp.dot(f, wl1[...], preferred_element_type=jnp.float32) + bl1r[...]
        al = jnp.dot(tl, wl2[...], preferred_element_type=jnp.float32) + bl2r[...]
        tr = jnp.dot(f, wr1[...], preferred_element_type=jnp.float32) + br1r[...]
        ar = jnp.dot(tr, wr2[...], preferred_element_type=jnp.float32) + br2r[...]
        so[...] = (al + ar) * 0.5

    return pl.pallas_call(
        body, grid=grid,
        in_specs=[row_spec, row_spec, den_spec, row_spec,
                  w_spec, b_spec, w2_spec, b2_spec,
                  w_spec, b_spec, w2_spec, b2_spec],
        out_specs=col_spec,
        out_shape=jax.ShapeDtypeStruct((NPAD, 1), jnp.float32),
    )(a0, a1, den, sk,
      Wl1, bl1.reshape(1, D), Wl2, bl2.reshape(1, 1),
      Wr1, br1.reshape(1, D), Wr2, br2.reshape(1, 1))


def _outer_call(s_row, s_col):
    BR = 256
    grid = (pl.cdiv(N, BR),)

    def body(sr, sc, o):
        z = sr[...] + sc[...]
        o[...] = 1.0 / (1.0 + jnp.exp(-z))

    return pl.pallas_call(
        body, grid=grid,
        in_specs=[pl.BlockSpec((BR, 1), lambda i: (i, 0)),
                  pl.BlockSpec((1, N), lambda i: (0, 0))],
        out_specs=pl.BlockSpec((BR, N), lambda i: (i, 0)),
        out_shape=jax.ShapeDtypeStruct((N, N), jnp.float32),
    )(s_row, s_col)


def _sc_logits_max(q, k, srcp, dstp, lane_iota):
    @functools.partial(
        pl.kernel,
        out_type=(jax.ShapeDtypeStruct((EPAD,), jnp.float32),
                  jax.ShapeDtypeStruct((NC * NPAD,), jnp.float32)),
        mesh=_mesh(),
        compiler_params=pltpu.CompilerParams(needs_layout_passes=False),
        scratch_types=[
            pltpu.VMEM((C,), jnp.int32),
            pltpu.VMEM((C,), jnp.int32),
            pltpu.VMEM((C,), jnp.int32),
            pltpu.VMEM((C,), jnp.int32),
            pltpu.VMEM((C, D), jnp.float32),
            pltpu.VMEM((C, D), jnp.float32),
            pltpu.VMEM((C, D), jnp.float32),
            pltpu.VMEM((C, D), jnp.float32),
            pltpu.VMEM((C,), jnp.float32),
            pltpu.VMEM((C,), jnp.float32),
            pltpu.VMEM((NPAD,), jnp.float32),
            pltpu.VMEM((NS, SLICE), jnp.float32),
            pltpu.VMEM((SLICE,), jnp.float32),
            pltpu.VMEM((L,), jnp.int32),
            pltpu.VMEM_SHARED((NS, NPAD), jnp.float32),
            pltpu.SemaphoreType.DMA,
            pltpu.SemaphoreType.DMA,
            pltpu.SemaphoreType.DMA,
            pltpu.SemaphoreType.DMA,
            pltpu.SemaphoreType.DMA,
            pltpu.SemaphoreType.DMA,
        ],
    )
    def kern(q_hbm, k_hbm, src_hbm, dst_hbm, li_hbm, lo_hbm, mp_hbm,
             dstb0, srcb0, dstb1, srcb1, qrows0, krows0, qrows1, krows1,
             lbuf0, lbuf1, mloc, mslice, outsl, lanesb, spm,
             msem0, msem1, gsem0, gsem1, wsem0, wsem1):
        cid = lax.axis_index("c")
        sid = lax.axis_index("s")
        ebase = jnp.where(cid == 0, sid * (CH0 * C),
                          NS * CH0 * C + sid * (CH1 * C))
        nch2 = jnp.where(cid == 0, CH0 // 2, CH1 // 2)

        slots = ((dstb0, srcb0, qrows0, krows0, lbuf0, msem0, gsem0, wsem0),
                 (dstb1, srcb1, qrows1, krows1, lbuf1, msem1, gsem1, wsem1))

        def meta_start(ch, si):
            dstb, srcb, _, _, _, msem, _, _ = slots[si]
            base = ebase + ch * C
            pltpu.async_copy(dst_hbm.at[pl.ds(base, C)], dstb, msem)
            pltpu.async_copy(src_hbm.at[pl.ds(base, C)], srcb, msem)

        def meta_wait(si):
            dstb, srcb, _, _, _, msem, _, _ = slots[si]
            pltpu.make_async_copy(dst_hbm.at[pl.ds(0, C)], dstb, msem).wait()
            pltpu.make_async_copy(src_hbm.at[pl.ds(0, C)], srcb, msem).wait()

        def gather_start(si):
            dstb, srcb, qrows, krows, _, _, gsem, _ = slots[si]
            pltpu.async_copy(q_hbm.at[dstb], qrows, gsem)
            pltpu.async_copy(k_hbm.at[srcb], krows, gsem)

        def gather_wait(si):
            dstb, srcb, qrows, krows, _, _, gsem, _ = slots[si]
            pltpu.make_async_copy(q_hbm.at[dstb], qrows, gsem).wait()
            pltpu.make_async_copy(k_hbm.at[srcb], krows, gsem).wait()

        pltpu.sync_copy(li_hbm, lanesb)
        neg = jnp.full((L,), NEG, jnp.float32)

        def initb(i, _):
            mloc[pl.ds(i * L, L)] = neg
            return 0
        lax.fori_loop(0, NPAD // L, initb, 0)

        lanes = lanesb[...]

        def process(ch, ch2, si):
            dstb, srcb, qrows, krows, lbuf, _, _, wsem = slots[si]
            base = ebase + ch * C

            @pl.when(ch2 > 0)
            def _():
                pltpu.make_async_copy(
                    lbuf, lo_hbm.at[pl.ds(0, C)], wsem).wait()

            def dot_grp(g, _):
                jb = g * L
                lv = jnp.zeros((L,), jnp.float32)
                for jj in range(L):
                    j = jb + jj
                    acc = qrows[j, pl.ds(0, L)] * krows[j, pl.ds(0, L)]
                    for cg in range(1, D // L):
                        acc = acc + (qrows[j, pl.ds(cg * L, L)] *
                                     krows[j, pl.ds(cg * L, L)])
                    s = _lane_sum(acc, lanes) * SCALE
                    lv = jnp.where(lanes == jj, s, lv)
                lbuf[pl.ds(jb, L)] = lv
                return 0
            lax.fori_loop(0, C // L, dot_grp, 0)

            pltpu.async_copy(lbuf, lo_hbm.at[pl.ds(base, C)], wsem)

            def grp(g, _):
                dv = dstb[pl.ds(g * L, L)]
                lv = lbuf[pl.ds(g * L, L)]
                acc = lv
                for r in range(1, L):
                    perm = (lanes + r) & (L - 1)
                    ki = jnp.take_along_axis(dv, perm, axis=0)
                    vi = jnp.take_along_axis(lv, perm, axis=0)
                    acc = jnp.where(dv == ki, jnp.maximum(acc, vi), acc)
                cur = plsc.load_gather(mloc, [dv])
                plsc.store_scatter(mloc, [dv], jnp.maximum(cur, acc))
                return 0
            lax.fori_loop(0, C // L, grp, 0)

        meta_start(0, 0)
        meta_wait(0)
        gather_start(0)
        meta_start(1, 1)

        def body2(ch2, _):
            c0 = 2 * ch2
            meta_wait(1)
            gather_start(1)
            gather_wait(0)
            process(c0, ch2, 0)

            @pl.when(ch2 + 1 < nch2)
            def _():
                meta_start(c0 + 2, 0)
                pltpu.make_async_copy(
                    dst_hbm.at[pl.ds(0, C)], dstb0, msem0).wait()
                pltpu.make_async_copy(
                    src_hbm.at[pl.ds(0, C)], srcb0, msem0).wait()
                gather_start(0)

            gather_wait(1)
            process(c0 + 1, ch2, 1)

            @pl.when(ch2 + 1 < nch2)
            def _():
                meta_start(c0 + 3, 1)
            return 0
        lax.fori_loop(0, nch2, body2, 0)
        pltpu.make_async_copy(lbuf0, lo_hbm.at[pl.ds(0, C)], wsem0).wait()
        pltpu.make_async_copy(lbuf1, lo_hbm.at[pl.ds(0, C)], wsem1).wait()

        pltpu.sync_copy(mloc, spm.at[sid])
        plsc.subcore_barrier()
        off = sid * SLICE
        for r in range(NS):
            pltpu.sync_copy(spm.at[r, pl.ds(off, SLICE)], mslice.at[r])

        def red(gi, _):
            a = mslice[0, pl.ds(gi * L, L)]
            for r in range(1, NS):
                a = jnp.maximum(a, mslice[r, pl.ds(gi * L, L)])
            outsl[pl.ds(gi * L, L)] = a
            return 0
        lax.fori_loop(0, SLICE // L, red, 0)
        pltpu.sync_copy(outsl, mp_hbm.at[pl.ds(cid * NPAD + off, SLICE)])

    return kern(q, k, srcp, dstp, lane_iota)


NACC = 10112
ASL = NACC // NS
CH0 = 112
CH1 = 2 * NCH - CH0


def _sc_softmax_agg(v, srcp, dstp, lo, mp):
    @functools.partial(
        pl.kernel,
        out_type=(jax.ShapeDtypeStruct((NW * NPAD,), jnp.float32),
                  jax.ShapeDtypeStruct((NC, NACC, D), jnp.float32)),
        mesh=_mesh(),
        compiler_params=pltpu.CompilerParams(needs_layout_passes=False),
        scratch_types=[
            pltpu.VMEM((2, C), jnp.int32),
            pltpu.VMEM((2, C), jnp.int32),
            pltpu.VMEM((C, D), jnp.float32),
            pltpu.VMEM((2, C), jnp.float32),
            pltpu.VMEM((2, C), jnp.float32),
            pltpu.VMEM((NPAD,), jnp.float32),
            pltpu.VMEM((NPAD,), jnp.float32),
            pltpu.VMEM((NPAD,), jnp.float32),
            pltpu.VMEM_SHARED((NACC, D), jnp.float32),
            pltpu.SemaphoreType.DMA,
            pltpu.SemaphoreType.DMA,
            pltpu.SemaphoreType.DMA,
            pltpu.SemaphoreType.DMA,
            pltpu.SemaphoreType.DMA,
            pltpu.SemaphoreType.DMA,
        ],
    )
    def kern(v_hbm, src_hbm, dst_hbm, lo_hbm, mp_hbm, dn_hbm, agg_hbm,
             dstb2, srcb2, vrows0, lbuf2, ebuf2,
             m0, m1, dloc, agg_spm,
             msem0, msem1, gsem0, gsem1, ssem0, ssem1):
        cid = lax.axis_index("c")
        sid = lax.axis_index("s")
        wid = cid * NS + sid
        ebase = jnp.where(cid == 0, sid * (CH0 * C),
                          NS * CH0 * C + sid * (CH1 * C))
        nch2 = jnp.where(cid == 0, CH0 // 2, CH1 // 2)

        slots = ((dstb2.at[0], srcb2.at[0], vrows0,
                  lbuf2.at[0], ebuf2.at[0], msem0, gsem0, ssem0),
                 (dstb2.at[1], srcb2.at[1], vrows0,
                  lbuf2.at[1], ebuf2.at[1], msem1, gsem1, ssem1))

        def meta_start(ch, si):
            dstb, srcb, _, lbuf, _, msem, _, _ = slots[si]
            base = ebase + ch * C
            pltpu.async_copy(dst_hbm.at[pl.ds(base, C)], dstb, msem)
            pltpu.async_copy(src_hbm.at[pl.ds(base, C)], srcb, msem)
            pltpu.async_copy(lo_hbm.at[pl.ds(base, C)], lbuf, msem)

        def meta_wait(si):
            dstb, srcb, _, lbuf, _, msem, _, _ = slots[si]
            pltpu.make_async_copy(dst_hbm.at[pl.ds(0, C)], dstb, msem).wait()
            pltpu.make_async_copy(src_hbm.at[pl.ds(0, C)], srcb, msem).wait()
            pltpu.make_async_copy(lo_hbm.at[pl.ds(0, C)], lbuf, msem).wait()

        def gather_start(si):
            _, srcb, vrows, _, _, _, gsem, _ = slots[si]
            pltpu.async_copy(v_hbm.at[srcb], vrows, gsem)

        def gather_wait(si):
            _, srcb, vrows, _, _, _, gsem, _ = slots[si]
            pltpu.make_async_copy(v_hbm.at[srcb], vrows, gsem).wait()

        def scatter_start(si):
            dstb, _, vrows, _, _, _, _, ssem = slots[si]
            pltpu.async_copy(vrows, agg_spm.at[dstb], ssem, add=True)

        def scatter_wait(si):
            dstb, _, vrows, _, _, _, _, ssem = slots[si]
            pltpu.make_async_copy(vrows, agg_spm.at[dstb], ssem).wait()

        pltpu.sync_copy(mp_hbm.at[pl.ds(0, NPAD)], m0)
        pltpu.sync_copy(mp_hbm.at[pl.ds(NPAD, NPAD)], m1)

        zer = jnp.zeros((L,), jnp.float32)

        def initd(i, _):
            dloc[pl.ds(i * L, L)] = zer
            return 0
        lax.fori_loop(0, NPAD // L, initd, 0)

        def initv(j, _):
            for cg in range(D // L):
                vrows0[j, pl.ds(cg * L, L)] = zer
            return 0
        lax.fori_loop(0, C, initv, 0)
        rb = sid * ASL
        for t in range(ASL // C):
            pltpu.sync_copy(vrows0, agg_spm.at[pl.ds(rb + t * C, C)])
        pltpu.sync_copy(vrows0.at[pl.ds(0, ASL % C)],
                        agg_spm.at[pl.ds(rb + (ASL // C) * C, ASL % C)])
        plsc.subcore_barrier()

        def grp_phase(si):
            dstb, _, _, lbuf, ebuf, _, _, _ = slots[si]

            def grp(g, _):
                dv = dstb[pl.ds(g * L, L)]
                lv = lbuf[pl.ds(g * L, L)]
                ma = jnp.maximum(plsc.load_gather(m0, [dv]),
                                 plsc.load_gather(m1, [dv]))
                ev = jnp.exp(lv - ma)
                ebuf[pl.ds(g * L, L)] = ev
                plsc.addupdate_scatter(dloc, [dv], ev)
                return 0
            lax.fori_loop(0, C // L, grp, 0)

        def srow_phase(si):
            _, _, vrows, _, ebuf, _, _, _ = slots[si]

            def srow_grp(g, _):
                jb = g * L
                ev = ebuf[pl.ds(jb, L)]
                for jj in range(L):
                    evs = ev[jj]
                    j = jb + jj
                    for cg in range(D // L):
                        vrows[j, pl.ds(cg * L, L)] = (
                            vrows[j, pl.ds(cg * L, L)] * evs)
                return 0
            lax.fori_loop(0, C // L, srow_grp, 0)

        meta_start(0, 0)
        meta_start(1, 1)

        def body2(ch2, _):
            c0 = 2 * ch2
            meta_wait(0)
            grp_phase(0)
            gather_start(0)
            gather_wait(0)
            srow_phase(0)
            scatter_start(0)

            meta_wait(1)
            grp_phase(1)

            @pl.when(ch2 + 1 < nch2)
            def _():
                meta_start(c0 + 2, 0)
            scatter_wait(0)
            gather_start(1)
            gather_wait(1)
            srow_phase(1)
            scatter_start(1)

            @pl.when(ch2 + 1 < nch2)
            def _():
                meta_start(c0 + 3, 1)
            scatter_wait(1)
            return 0
        lax.fori_loop(0, nch2, body2, 0)

        plsc.subcore_barrier()
        pltpu.sync_copy(agg_spm.at[pl.ds(rb, ASL)],
                        agg_hbm.at[cid, pl.ds(rb, ASL)])

        pltpu.sync_copy(dloc, dn_hbm.at[pl.ds(wid * NPAD, NPAD)])

    return kern(v, srcp, dstp, lo, mp)


def kernel(x, edge_index, Wq1, bq1, Wk1, bk1, Wv1, bv1, Ws1, bs1,
           Wq2, bq2, Wk2, bk2, Wv2, bv2, Ws2, bs2,
           Wl1, bl1, Wl2, bl2, Wr1, br1, Wr2, br2):
    x = jax.lax.stop_gradient(x)
    src = edge_index[0].astype(jnp.int32)
    dst = edge_index[1].astype(jnp.int32)
    pad_idx = jnp.full((EPAD - E,), N, jnp.int32)
    srcp = jnp.concatenate([src, pad_idx])
    dstp = jnp.concatenate([dst, pad_idx])
    xp = jnp.pad(x, ((0, NPAD - N), (0, 0)))
    lane_iota = jnp.arange(L, dtype=jnp.int32)

    q1, k1, v1, s1 = _proj4_call(xp, Wq1, bq1, Wk1, bk1, Wv1, bv1, Ws1, bs1)
    lo1, mp1 = _sc_logits_max(q1, k1, srcp, dstp, lane_iota)
    dp1, ap1 = _sc_softmax_agg(v1, srcp, dstp, lo1, mp1)

    def pad_acc(a):
        return jnp.pad(a, ((0, 0), (0, NPAD - NACC), (0, 0)))

    ap1p = pad_acc(ap1)
    q2, k2, v2, s2 = _proj4_call(
        s1, Wq2, bq2, Wk2, bk2, Wv2, bv2, Ws2, bs2, relu_in=True,
        a0=ap1p[0], a1=ap1p[1], den=dp1.reshape(NW, NPAD).T)
    lo2, mp2 = _sc_logits_max(q2, k2, srcp, dstp, lane_iota)
    dp2, ap2 = _sc_softmax_agg(v2, srcp, dstp, lo2, mp2)

    ap2p = pad_acc(ap2)
    s_vec = _finalize_call(
        ap2p[0], ap2p[1], dp2.reshape(NW, NPAD).T,
        s2, Wl1, bl1, Wl2, bl2, Wr1, br1, Wr2, br2)

    s_n = s_vec[:N]
    return _outer_call(s_n, s_n.reshape(1, N))
